# baseline (speedup 1.0000x reference)
import jax
import jax.numpy as jnp
from jax import lax
from jax.experimental import pallas as pl
from jax.experimental.pallas import tpu as pltpu

_HIDDEN = 2048
_EXPERTS = 64
_TC = 1024
_NBUF = 3


def _outer(x_hbm, w_ref, o_hbm):
    def _inner(x_blk, o_blk):
        o_blk[...] = lax.dot_general(
            x_blk[...],
            w_ref[...],
            dimension_numbers=(((1,), (1,)), ((), ())),
            preferred_element_type=jnp.float32,
        )

    nchunks = x_hbm.shape[0] // _TC
    pltpu.emit_pipeline(
        _inner,
        grid=(nchunks,),
        in_specs=[
            pl.BlockSpec(
                (_TC, _HIDDEN),
                lambda i: (i, 0),
                pipeline_mode=pl.Buffered(buffer_count=_NBUF),
            )
        ],
        out_specs=[pl.BlockSpec((_TC, _EXPERTS), lambda i: (i, 0))],
    )(x_hbm, o_hbm)


def kernel(hidden_states, weight):
    hs = hidden_states.reshape(-1, _HIDDEN)
    m = hs.shape[0]
    out = pl.pallas_call(
        _outer,
        in_specs=[
            pl.BlockSpec(memory_space=pltpu.MemorySpace.HBM),
            pl.BlockSpec(memory_space=pltpu.MemorySpace.VMEM),
        ],
        out_specs=pl.BlockSpec(memory_space=pltpu.MemorySpace.HBM),
        out_shape=jax.ShapeDtypeStruct((m, _EXPERTS), jnp.float32),
    )(hs, weight)
    return out
